# baseline (speedup 1.0000x reference)
import jax
import jax.numpy as jnp
from jax.experimental import pallas as pl

N = 10000
E = 160000
H = 128
L = 4


def _ssp(v):
    return jax.nn.softplus(v) - jnp.log(2.0)


def _readout_body(h_ref, wfc_ref, bfc_ref, wfc2_ref, bfc2_ref, o_ref):
    m = jnp.sum(h_ref[...], axis=0, keepdims=True) * (1.0 / N)
    y = jnp.dot(m, wfc_ref[...], preferred_element_type=jnp.float32) + bfc_ref[...]
    o_ref[...] = (
        jnp.dot(y, wfc2_ref[...], preferred_element_type=jnp.float32) + bfc2_ref[...]
    )


def kernel(x, z, edge_index, d, emb1, emb2, Wf1, bf1, Wf2, bf2, Wo, bo, Wfc, bfc, Wfc2, bfc2):
    src = edge_index[0]
    dst = edge_index[1]
    centers = jnp.linspace(0.0, 1.0, H)
    gamma = 1.0 / (centers[1] - centers[0]) ** 2
    bf = jnp.exp(-gamma * (d[:, None] - centers[None, :]) ** 2)
    co = jnp.where(d < 1.0, 0.5 * (jnp.cos(jnp.pi * d) + 1.0), 0.0)[:, None]
    bf = bf * co
    h_edge = emb1[z[src]] + emb2[z[dst]]

    def conv(i, h_nodes):
        filt = _ssp(bf @ Wf1[i] + bf1[i]) @ Wf2[i] + bf2[i]
        msg = h_nodes[src] * filt * h_edge * co
        agg = jax.ops.segment_sum(msg, dst, num_segments=N)
        return _ssp(agg @ Wo[i] + bo[i])

    h = conv(0, x)
    for i in range(1, L):
        h = h + conv(i, h)

    y = pl.pallas_call(
        _readout_body,
        out_shape=jax.ShapeDtypeStruct((1, 1), jnp.float32),
    )(h, Wfc, bfc.reshape(1, H), Wfc2, bfc2.reshape(1, 1))
    return y


# speedup vs baseline: 3.3419x; 3.3419x over previous
import functools

import jax
import jax.numpy as jnp
import numpy as np
from jax import lax
from jax.experimental import pallas as pl
from jax.experimental.pallas import tpu as pltpu
from jax.experimental.pallas import tpu_sc as plsc

N = 10000
E = 160000
H = 128
L = 4
NCOLORS = 100

NC = 2
NS = 16
TILES = NC * NS
EPT = E // TILES
CH = 104
NFULL = EPT // CH
TAIL = EPT - NFULL * CH
NPT = 624
ZR = 104

_LN2 = float(np.log(np.float32(2.0)))
_PI = float(np.float32(np.pi))

_mesh = plsc.VectorSubcoreMesh(core_axis_name="c", subcore_axis_name="s")
_f32 = jnp.float32


def _ssp(v):
    return jnp.maximum(v, 0.0) + jnp.log1p(jnp.exp(-jnp.abs(v))) - _LN2


def _ew(dst_ref, a_ref, b_ref, n, op):
    @pl.loop(0, n)
    def _(r):
        for g in range(H // 16):
            slc = (pl.ds(r, 1), pl.ds(g * 16, 16))
            if op == "add":
                dst_ref.at[*slc][...] = a_ref.at[*slc][...] + b_ref.at[*slc][...]
            else:
                dst_ref.at[*slc][...] = a_ref.at[*slc][...] * b_ref.at[*slc][...]


def _he_body(src_hbm, dst_hbm, a_hbm, b_hbm, he_hbm,
             sv, dv, av, bv, sv8, dv8, av8, bv8):
    w = lax.axis_index("c") * NS + lax.axis_index("s")
    base0 = w * EPT

    def do_chunk(base, n, sv, dv, av, bv):
        pltpu.sync_copy(src_hbm.at[pl.ds(base, n)], sv)
        pltpu.sync_copy(dst_hbm.at[pl.ds(base, n)], dv)
        pltpu.sync_copy(a_hbm.at[sv], av)
        pltpu.sync_copy(b_hbm.at[dv], bv)
        _ew(av, av, bv, n, "add")
        pltpu.sync_copy(av, he_hbm.at[pl.ds(base, n)])

    @pl.loop(0, NFULL)
    def _(j):
        do_chunk(base0 + j * CH, CH, sv, dv, av, bv)

    do_chunk(base0 + NFULL * CH, TAIL, sv8, dv8, av8, bv8)


_he_call = functools.partial(
    pl.kernel,
    out_type=jax.ShapeDtypeStruct((E, H), _f32),
    mesh=_mesh,
    scratch_types=[
        pltpu.VMEM((CH,), jnp.int32), pltpu.VMEM((CH,), jnp.int32),
        pltpu.VMEM((CH, H), _f32), pltpu.VMEM((CH, H), _f32),
        pltpu.VMEM((TAIL,), jnp.int32), pltpu.VMEM((TAIL,), jnp.int32),
        pltpu.VMEM((TAIL, H), _f32), pltpu.VMEM((TAIL, H), _f32),
    ],
)(_he_body)


def _conv_body(h_hbm, src_hbm, dst_hbm, co_hbm, p0_hbm, p1_hbm,
               sv, dv, cv, hv, sv8, dv8, cv8, hv8, zb, acc):
    core = lax.axis_index("c")
    s = lax.axis_index("s")
    w = core * NS + s

    @pl.loop(0, ZR)
    def _(r):
        for g in range(H // 16):
            zb.at[pl.ds(r, 1), pl.ds(g * 16, 16)][...] = jnp.zeros((1, 16), _f32)

    for k in range(NPT // ZR):
        pltpu.sync_copy(zb, acc.at[pl.ds(s * NPT + k * ZR, ZR)])

    @pl.when(s == NS - 1)
    def _():
        pltpu.sync_copy(zb.at[pl.ds(0, 16)], acc.at[pl.ds(NS * NPT, 16)])

    plsc.subcore_barrier()

    base0 = w * EPT

    def do_chunk(base, n, sv, dv, cv, hv):
        pltpu.sync_copy(src_hbm.at[pl.ds(base, n)], sv)
        pltpu.sync_copy(dst_hbm.at[pl.ds(base, n)], dv)
        pltpu.sync_copy(co_hbm.at[pl.ds(base, n)], cv)
        pltpu.sync_copy(h_hbm.at[sv], hv)
        _ew(hv, hv, cv, n, "mul")
        pltpu.sync_copy(hv, acc.at[dv], add=True)

    @pl.loop(0, NFULL)
    def _(j):
        do_chunk(base0 + j * CH, CH, sv, dv, cv, hv)

    do_chunk(base0 + NFULL * CH, TAIL, sv8, dv8, cv8, hv8)
    plsc.subcore_barrier()

    def wb(rs, nr):
        @pl.when(core == 0)
        def _():
            pltpu.sync_copy(acc.at[pl.ds(rs, nr)], p0_hbm.at[pl.ds(rs, nr)])

        @pl.when(core == 1)
        def _():
            pltpu.sync_copy(acc.at[pl.ds(rs, nr)], p1_hbm.at[pl.ds(rs, nr)])

    for k in range(NPT // ZR):
        wb(s * NPT + k * ZR, ZR)

    @pl.when(s == NS - 1)
    def _():
        wb(NS * NPT, 16)


_conv_call = functools.partial(
    pl.kernel,
    out_type=[jax.ShapeDtypeStruct((N, H), _f32),
              jax.ShapeDtypeStruct((N, H), _f32)],
    mesh=_mesh,
    scratch_types=[
        pltpu.VMEM((CH,), jnp.int32), pltpu.VMEM((CH,), jnp.int32),
        pltpu.VMEM((CH, H), _f32), pltpu.VMEM((CH, H), _f32),
        pltpu.VMEM((TAIL,), jnp.int32), pltpu.VMEM((TAIL,), jnp.int32),
        pltpu.VMEM((TAIL, H), _f32), pltpu.VMEM((TAIL, H), _f32),
        pltpu.VMEM((ZR, H), _f32),
        pltpu.VMEM_SHARED((N, H), _f32),
    ],
)(_conv_body)


def _ab_body(z_ref, e1_ref, e2_ref, a_ref, b_ref):
    oh = (z_ref[...] == lax.broadcasted_iota(jnp.int32, (1, NCOLORS), 1)
          ).astype(_f32)
    a_ref[...] = jnp.dot(oh, e1_ref[...], preferred_element_type=_f32)
    b_ref[...] = jnp.dot(oh, e2_ref[...], preferred_element_type=_f32)


def _ab(z2, emb1, emb2):
    return pl.pallas_call(
        _ab_body,
        out_shape=[jax.ShapeDtypeStruct((N, H), _f32),
                   jax.ShapeDtypeStruct((N, H), _f32)],
    )(z2, emb1, emb2)


EB = 2000


def _coeff_body(gamma, d_ref, cen_ref, he_ref, w1_ref, b1_ref, w2_ref, b2_ref,
                c_ref):
    dd = d_ref[...]
    diff = dd - cen_ref[...]
    bf = jnp.exp(-gamma * (diff * diff))
    co = jnp.where(dd < 1.0, 0.5 * (jnp.cos(_PI * dd) + 1.0), 0.0)
    bfc = bf * co
    t = _ssp(jnp.dot(bfc, w1_ref[...], preferred_element_type=_f32)
             + b1_ref[...])
    filt = jnp.dot(t, w2_ref[...], preferred_element_type=_f32) + b2_ref[...]
    c_ref[...] = filt * he_ref[...] * co


def _coeff(gamma, d2, cen, he, w1, b1, w2, b2):
    blk = lambda i: (i, 0)
    full = lambda i: (0, 0)
    return pl.pallas_call(
        functools.partial(_coeff_body, gamma),
        grid=(E // EB,),
        in_specs=[
            pl.BlockSpec((EB, 1), blk),
            pl.BlockSpec((1, H), full),
            pl.BlockSpec((EB, H), blk),
            pl.BlockSpec((H, H), full),
            pl.BlockSpec((1, H), full),
            pl.BlockSpec((H, H), full),
            pl.BlockSpec((1, H), full),
        ],
        out_specs=pl.BlockSpec((EB, H), blk),
        out_shape=jax.ShapeDtypeStruct((E, H), _f32),
    )(d2, cen, he, w1, b1, w2, b2)


NB = 2000


def _post_body(res, p0_ref, p1_ref, wo_ref, bo_ref, *rest):
    if res:
        h_ref, o_ref = rest
    else:
        (o_ref,) = rest
    agg = p0_ref[...] + p1_ref[...]
    o = _ssp(jnp.dot(agg, wo_ref[...], preferred_element_type=_f32)
             + bo_ref[...])
    if res:
        o = o + h_ref[...]
    o_ref[...] = o


def _post(p0, p1, wo, bo, hprev):
    res = hprev is not None
    blk = lambda i: (i, 0)
    full = lambda i: (0, 0)
    in_specs = [
        pl.BlockSpec((NB, H), blk),
        pl.BlockSpec((NB, H), blk),
        pl.BlockSpec((H, H), full),
        pl.BlockSpec((1, H), full),
    ]
    args = [p0, p1, wo, bo]
    if res:
        in_specs.append(pl.BlockSpec((NB, H), blk))
        args.append(hprev)
    return pl.pallas_call(
        functools.partial(_post_body, res),
        grid=(N // NB,),
        in_specs=in_specs,
        out_specs=pl.BlockSpec((NB, H), blk),
        out_shape=jax.ShapeDtypeStruct((N, H), _f32),
    )(*args)


def _readout_body(h_ref, wfc_ref, bfc_ref, wfc2_ref, bfc2_ref, o_ref):
    yn = jnp.dot(h_ref[...], wfc_ref[...], preferred_element_type=_f32) \
        + bfc_ref[...]
    m = jnp.mean(yn, axis=0, keepdims=True)
    o_ref[...] = jnp.dot(m, wfc2_ref[...], preferred_element_type=_f32) \
        + bfc2_ref[...]


def _readout(h, wfc, bfc2d, wfc2, bfc2):
    return pl.pallas_call(
        _readout_body,
        out_shape=jax.ShapeDtypeStruct((1, 1), _f32),
    )(h, wfc, bfc2d, wfc2, bfc2)


def kernel(x, z, edge_index, d, emb1, emb2, Wf1, bf1, Wf2, bf2, Wo, bo,
           Wfc, bfc, Wfc2, bfc2):
    src = edge_index[0]
    dst = edge_index[1]
    d2 = d.reshape(E, 1)
    z2 = z.astype(jnp.int32).reshape(N, 1)

    centers = np.concatenate([
        np.arange(H - 1, dtype=np.float32) / np.float32(H - 1),
        np.ones((1,), np.float32),
    ])
    gamma = float(np.float32(1.0) / (centers[1] - centers[0]) ** 2)
    cen = jnp.asarray(centers.reshape(1, H))

    a_nodes, b_nodes = _ab(z2, emb1, emb2)
    he = _he_call(src, dst, a_nodes, b_nodes)

    coeffs = [
        _coeff(gamma, d2, cen, he, Wf1[i], bf1[i].reshape(1, H),
               Wf2[i], bf2[i].reshape(1, H))
        for i in range(L)
    ]

    h = x
    for i in range(L):
        p0, p1 = _conv_call(h, src, dst, coeffs[i])
        h = _post(p0, p1, Wo[i], bo[i].reshape(1, H), None if i == 0 else h)

    return _readout(h, Wfc, bfc.reshape(1, H), Wfc2, bfc2.reshape(1, 1))


# speedup vs baseline: 3.8025x; 1.1378x over previous
import functools

import jax
import jax.numpy as jnp
import numpy as np
from jax import lax
from jax.experimental import pallas as pl
from jax.experimental.pallas import tpu as pltpu
from jax.experimental.pallas import tpu_sc as plsc

N = 10000
E = 160000
H = 128
L = 4
NCOLORS = 100

NC = 2
NS = 16
TILES = NC * NS
EPT = E // TILES
CHE = 104
NFE = EPT // CHE
CHC = 64
NFC = EPT // CHC
TAIL = 8
NPT = 624
ZR = 104

_LN2 = float(np.log(np.float32(2.0)))
_PI = float(np.float32(np.pi))

_mesh = plsc.VectorSubcoreMesh(core_axis_name="c", subcore_axis_name="s")
_f32 = jnp.float32


def _ssp(v):
    return jnp.maximum(v, 0.0) + jnp.log1p(jnp.exp(-jnp.abs(v))) - _LN2


def _ew(dst_ref, a_ref, b_ref, n, op):
    @pl.loop(0, n)
    def _(r):
        for g in range(H // 16):
            slc = (pl.ds(r, 1), pl.ds(g * 16, 16))
            if op == "add":
                dst_ref.at[*slc][...] = a_ref.at[*slc][...] + b_ref.at[*slc][...]
            else:
                dst_ref.at[*slc][...] = a_ref.at[*slc][...] * b_ref.at[*slc][...]


def _he_body(src_hbm, dst_hbm, a_hbm, b_hbm, he_hbm,
             sv, dv, av, bv, sv8, dv8, av8, bv8, ss, ds, ga, gb, os):
    w = lax.axis_index("c") * NS + lax.axis_index("s")
    base0 = w * EPT
    base = lambda j: base0 + j * CHE

    def idx_issue(j, b):
        pltpu.async_copy(src_hbm.at[pl.ds(base(j), CHE)], sv[b], ss[b])
        pltpu.async_copy(dst_hbm.at[pl.ds(base(j), CHE)], dv[b], ds[b])

    def idx_wait(j, b):
        pltpu.make_async_copy(src_hbm.at[pl.ds(base(j), CHE)], sv[b], ss[b]).wait()
        pltpu.make_async_copy(dst_hbm.at[pl.ds(base(j), CHE)], dv[b], ds[b]).wait()

    def gather_issue(b):
        pltpu.async_copy(a_hbm.at[sv[b]], av[b], ga[b])
        pltpu.async_copy(b_hbm.at[dv[b]], bv[b], gb[b])

    def store_wait(j, b):
        pltpu.make_async_copy(av[b], he_hbm.at[pl.ds(base(j), CHE)], os[b]).wait()

    idx_issue(0, 0)
    idx_wait(0, 0)
    gather_issue(0)
    idx_issue(1, 1)

    @pl.loop(0, NFE, step=2)
    def _(j0):
        for b in range(2):
            j = j0 + b
            q = 1 - b

            @pl.when(j + 1 < NFE)
            def _():
                idx_wait(j + 1, q)

                @pl.when(j >= 1)
                def _():
                    store_wait(j - 1, q)

                gather_issue(q)

            pltpu.make_async_copy(a_hbm.at[sv[b]], av[b], ga[b]).wait()
            pltpu.make_async_copy(b_hbm.at[dv[b]], bv[b], gb[b]).wait()
            _ew(av[b], av[b], bv[b], CHE, "add")
            pltpu.async_copy(av[b], he_hbm.at[pl.ds(base(j), CHE)], os[b])

            @pl.when(j + 2 < NFE)
            def _():
                idx_issue(j + 2, b)

    store_wait(NFE - 2, 0)
    store_wait(NFE - 1, 1)

    tb = base0 + NFE * CHE
    pltpu.sync_copy(src_hbm.at[pl.ds(tb, TAIL)], sv8)
    pltpu.sync_copy(dst_hbm.at[pl.ds(tb, TAIL)], dv8)
    pltpu.sync_copy(a_hbm.at[sv8], av8)
    pltpu.sync_copy(b_hbm.at[dv8], bv8)
    _ew(av8, av8, bv8, TAIL, "add")
    pltpu.sync_copy(av8, he_hbm.at[pl.ds(tb, TAIL)])


_he_call = functools.partial(
    pl.kernel,
    out_type=jax.ShapeDtypeStruct((E, H), _f32),
    mesh=_mesh,
    scratch_types=[
        [pltpu.VMEM((CHE,), jnp.int32)] * 2,
        [pltpu.VMEM((CHE,), jnp.int32)] * 2,
        [pltpu.VMEM((CHE, H), _f32)] * 2,
        [pltpu.VMEM((CHE, H), _f32)] * 2,
        pltpu.VMEM((TAIL,), jnp.int32), pltpu.VMEM((TAIL,), jnp.int32),
        pltpu.VMEM((TAIL, H), _f32), pltpu.VMEM((TAIL, H), _f32),
        [pltpu.SemaphoreType.DMA] * 2, [pltpu.SemaphoreType.DMA] * 2,
        [pltpu.SemaphoreType.DMA] * 2, [pltpu.SemaphoreType.DMA] * 2,
        [pltpu.SemaphoreType.DMA] * 2,
    ],
)(_he_body)


def _conv_body(h_hbm, src_hbm, dst_hbm, co_hbm, p0_hbm, p1_hbm,
               sv, dv, cv, hv, sv8, dv8, cv8, hv8, acc,
               ss, ds, cs, gs, ws):
    core = lax.axis_index("c")
    s = lax.axis_index("s")
    w = core * NS + s

    @pl.loop(0, CHC)
    def _(r):
        for g in range(H // 16):
            cv[0].at[pl.ds(r, 1), pl.ds(g * 16, 16)][...] = \
                jnp.zeros((1, 16), _f32)

    for k in range(NPT // CHC):
        pltpu.sync_copy(cv[0], acc.at[pl.ds(s * NPT + k * CHC, CHC)])
    pltpu.sync_copy(cv[0].at[pl.ds(0, NPT % CHC)],
                    acc.at[pl.ds(s * NPT + (NPT // CHC) * CHC, NPT % CHC)])

    @pl.when(s == NS - 1)
    def _():
        pltpu.sync_copy(cv[0].at[pl.ds(0, 16)], acc.at[pl.ds(NS * NPT, 16)])

    plsc.subcore_barrier()

    base0 = w * EPT
    base = lambda j: base0 + j * CHC

    def idx_issue(j, b):
        pltpu.async_copy(src_hbm.at[pl.ds(base(j), CHC)], sv[b], ss[b])
        pltpu.async_copy(dst_hbm.at[pl.ds(base(j), CHC)], dv[b], ds[b])
        pltpu.async_copy(co_hbm.at[pl.ds(base(j), CHC)], cv[b], cs[b])

    def scat_wait(b):
        pltpu.make_async_copy(hv[b], acc.at[dv[b]], ws[b]).wait()

    idx_issue(0, 0)
    pltpu.make_async_copy(src_hbm.at[pl.ds(base(0), CHC)], sv[0], ss[0]).wait()
    pltpu.async_copy(h_hbm.at[sv[0]], hv[0], gs[0])
    idx_issue(1, 1)

    @pl.loop(0, NFC, step=2)
    def _(j0):
        for b in range(2):
            j = j0 + b
            q = 1 - b

            @pl.when(j + 1 < NFC)
            def _():
                pltpu.make_async_copy(
                    src_hbm.at[pl.ds(base(j + 1), CHC)], sv[q], ss[q]).wait()

                @pl.when(j >= 1)
                def _():
                    scat_wait(q)

                pltpu.async_copy(h_hbm.at[sv[q]], hv[q], gs[q])

            pltpu.make_async_copy(h_hbm.at[sv[b]], hv[b], gs[b]).wait()
            pltpu.make_async_copy(
                co_hbm.at[pl.ds(base(j), CHC)], cv[b], cs[b]).wait()
            _ew(hv[b], hv[b], cv[b], CHC, "mul")
            pltpu.make_async_copy(
                dst_hbm.at[pl.ds(base(j), CHC)], dv[b], ds[b]).wait()
            pltpu.async_copy(hv[b], acc.at[dv[b]], ws[b], add=True)

            @pl.when(j + 2 < NFC)
            def _():
                idx_issue(j + 2, b)

    scat_wait(0)
    scat_wait(1)

    tb = base0 + NFC * CHC
    pltpu.sync_copy(src_hbm.at[pl.ds(tb, TAIL)], sv8)
    pltpu.sync_copy(dst_hbm.at[pl.ds(tb, TAIL)], dv8)
    pltpu.sync_copy(co_hbm.at[pl.ds(tb, TAIL)], cv8)
    pltpu.sync_copy(h_hbm.at[sv8], hv8)
    _ew(hv8, hv8, cv8, TAIL, "mul")
    pltpu.sync_copy(hv8, acc.at[dv8], add=True)
    plsc.subcore_barrier()

    def wb(rs, nr):
        @pl.when(core == 0)
        def _():
            pltpu.sync_copy(acc.at[pl.ds(rs, nr)], p0_hbm.at[pl.ds(rs, nr)])

        @pl.when(core == 1)
        def _():
            pltpu.sync_copy(acc.at[pl.ds(rs, nr)], p1_hbm.at[pl.ds(rs, nr)])

    for k in range(NPT // ZR):
        wb(s * NPT + k * ZR, ZR)

    @pl.when(s == NS - 1)
    def _():
        wb(NS * NPT, 16)


_conv_call = functools.partial(
    pl.kernel,
    out_type=[jax.ShapeDtypeStruct((N, H), _f32),
              jax.ShapeDtypeStruct((N, H), _f32)],
    mesh=_mesh,
    scratch_types=[
        [pltpu.VMEM((CHC,), jnp.int32)] * 2,
        [pltpu.VMEM((CHC,), jnp.int32)] * 2,
        [pltpu.VMEM((CHC, H), _f32)] * 2,
        [pltpu.VMEM((CHC, H), _f32)] * 2,
        pltpu.VMEM((TAIL,), jnp.int32), pltpu.VMEM((TAIL,), jnp.int32),
        pltpu.VMEM((TAIL, H), _f32), pltpu.VMEM((TAIL, H), _f32),
        pltpu.VMEM_SHARED((N, H), _f32),
        [pltpu.SemaphoreType.DMA] * 2, [pltpu.SemaphoreType.DMA] * 2,
        [pltpu.SemaphoreType.DMA] * 2, [pltpu.SemaphoreType.DMA] * 2,
        [pltpu.SemaphoreType.DMA] * 2,
    ],
)(_conv_body)


def _ab_body(z_ref, e1_ref, e2_ref, a_ref, b_ref):
    oh = (z_ref[...] == lax.broadcasted_iota(jnp.int32, (1, NCOLORS), 1)
          ).astype(_f32)
    a_ref[...] = jnp.dot(oh, e1_ref[...], preferred_element_type=_f32,
            precision=lax.Precision.HIGHEST)
    b_ref[...] = jnp.dot(oh, e2_ref[...], preferred_element_type=_f32,
            precision=lax.Precision.HIGHEST)


def _ab(z2, emb1, emb2):
    return pl.pallas_call(
        _ab_body,
        out_shape=[jax.ShapeDtypeStruct((N, H), _f32),
                   jax.ShapeDtypeStruct((N, H), _f32)],
    )(z2, emb1, emb2)


EB = 2000


def _coeff_body(d_ref, cen_ref, ng_ref, he_ref, w1_ref, b1_ref, w2_ref, b2_ref,
                c_ref):
    dd = d_ref[...]
    diff = dd - cen_ref[...]
    bf = jnp.exp(ng_ref[...] * (diff * diff))
    co = jnp.where(dd < 1.0, 0.5 * (jnp.cos(_PI * dd) + 1.0), 0.0)
    bfc = bf * co
    t = _ssp(jnp.dot(bfc, w1_ref[...].astype(jnp.bfloat16),
                     preferred_element_type=_f32) + b1_ref[...])
    filt = jnp.dot(t.astype(jnp.bfloat16), w2_ref[...].astype(jnp.bfloat16),
                   preferred_element_type=_f32) + b2_ref[...]
    c_ref[...] = filt * he_ref[...] * co


def _coeff(d2, cen, ng, he, w1, b1, w2, b2):
    blk = lambda i: (i, 0)
    full = lambda i: (0, 0)
    return pl.pallas_call(
        _coeff_body,
        grid=(E // EB,),
        in_specs=[
            pl.BlockSpec((EB, 1), blk),
            pl.BlockSpec((1, H), full),
            pl.BlockSpec((1, 1), full),
            pl.BlockSpec((EB, H), blk),
            pl.BlockSpec((H, H), full),
            pl.BlockSpec((1, H), full),
            pl.BlockSpec((H, H), full),
            pl.BlockSpec((1, H), full),
        ],
        out_specs=pl.BlockSpec((EB, H), blk),
        out_shape=jax.ShapeDtypeStruct((E, H), _f32),
    )(d2, cen, ng, he, w1, b1, w2, b2)


NB = 2000


def _post_body(res, p0_ref, p1_ref, wo_ref, bo_ref, *rest):
    if res:
        h_ref, o_ref = rest
    else:
        (o_ref,) = rest
    agg = p0_ref[...] + p1_ref[...]
    o = _ssp(jnp.dot(agg, wo_ref[...].astype(jnp.bfloat16),
                     preferred_element_type=_f32) + bo_ref[...])
    if res:
        o = o + h_ref[...]
    o_ref[...] = o


def _post(p0, p1, wo, bo, hprev):
    res = hprev is not None
    blk = lambda i: (i, 0)
    full = lambda i: (0, 0)
    in_specs = [
        pl.BlockSpec((NB, H), blk),
        pl.BlockSpec((NB, H), blk),
        pl.BlockSpec((H, H), full),
        pl.BlockSpec((1, H), full),
    ]
    args = [p0, p1, wo, bo]
    if res:
        in_specs.append(pl.BlockSpec((NB, H), blk))
        args.append(hprev)
    return pl.pallas_call(
        functools.partial(_post_body, res),
        grid=(N // NB,),
        in_specs=in_specs,
        out_specs=pl.BlockSpec((NB, H), blk),
        out_shape=jax.ShapeDtypeStruct((N, H), _f32),
    )(*args)


def _readout_body(h_ref, wfc_ref, bfc_ref, wfc2_ref, bfc2_ref, o_ref):
    yn = jnp.dot(h_ref[...].astype(jnp.bfloat16), wfc_ref[...],
                 preferred_element_type=_f32) + bfc_ref[...]
    m = jnp.sum(yn, axis=0, keepdims=True) * np.float32(1.0 / N)
    o_ref[...] = jnp.sum(m * wfc2_ref[...], axis=1, keepdims=True) \
        + bfc2_ref[...]


def _readout(h, wfc, bfc2d, wfc2row, bfc2):
    return pl.pallas_call(
        _readout_body,
        out_shape=jax.ShapeDtypeStruct((1, 1), _f32),
    )(h, wfc, bfc2d, wfc2row, bfc2)


def kernel(x, z, edge_index, d, emb1, emb2, Wf1, bf1, Wf2, bf2, Wo, bo,
           Wfc, bfc, Wfc2, bfc2):
    src = edge_index[0]
    dst = edge_index[1]
    d2 = d.reshape(E, 1)
    z2 = z.astype(jnp.int32).reshape(N, 1)

    centers = jnp.linspace(0.0, 1.0, H)
    gamma = 1.0 / (centers[1] - centers[0]) ** 2
    cen = centers.reshape(1, H)
    ng = (-gamma).reshape(1, 1).astype(_f32)

    a_nodes, b_nodes = _ab(z2, emb1, emb2)
    he = _he_call(src, dst, a_nodes, b_nodes)

    coeffs = [
        _coeff(d2, cen, ng, he, Wf1[i], bf1[i].reshape(1, H),
               Wf2[i], bf2[i].reshape(1, H))
        for i in range(L)
    ]

    h = x
    for i in range(L):
        p0, p1 = _conv_call(h, src, dst, coeffs[i])
        h = _post(p0, p1, Wo[i], bo[i].reshape(1, H), None if i == 0 else h)

    return _readout(h, Wfc, bfc.reshape(1, H), Wfc2.reshape(1, H),
                    bfc2.reshape(1, 1))


# speedup vs baseline: 5.8501x; 1.5385x over previous
import functools

import jax
import jax.numpy as jnp
import numpy as np
from jax import lax
from jax.experimental import pallas as pl
from jax.experimental.pallas import tpu as pltpu
from jax.experimental.pallas import tpu_sc as plsc

N = 10000
E = 160000
H = 128
L = 4
NCOLORS = 100

NC = 2
NS = 16
TILES = NC * NS
EPT = E // TILES
CHE = 104
NFE = EPT // CHE
CHC = 80
NFC = EPT // CHC
TAILE = 8
TAILC = 40
NPT = 624
ZR = 104

_LN2 = float(np.log(np.float32(2.0)))
_PI = float(np.float32(np.pi))

_mesh = plsc.VectorSubcoreMesh(core_axis_name="c", subcore_axis_name="s")
_f32 = jnp.float32


def _ssp(v):
    return jnp.maximum(v, 0.0) + jnp.log1p(jnp.exp(-jnp.abs(v))) - _LN2


def _ew(dst_ref, a_ref, b_ref, n, op):
    @pl.loop(0, n)
    def _(r):
        for g in range(H // 16):
            slc = (pl.ds(r, 1), pl.ds(g * 16, 16))
            if op == "add":
                dst_ref.at[*slc][...] = a_ref.at[*slc][...] + b_ref.at[*slc][...]
            else:
                dst_ref.at[*slc][...] = a_ref.at[*slc][...] * b_ref.at[*slc][...]


def _he_body(src_hbm, dst_hbm, a_hbm, b_hbm, he_hbm,
             sv, dv, av, bv, sv8, dv8, av8, bv8, ss, ds, ga, gb, os):
    w = lax.axis_index("c") * NS + lax.axis_index("s")
    base0 = w * EPT
    base = lambda j: base0 + j * CHE

    def idx_issue(j, b):
        pltpu.async_copy(src_hbm.at[pl.ds(base(j), CHE)], sv[b], ss[b])
        pltpu.async_copy(dst_hbm.at[pl.ds(base(j), CHE)], dv[b], ds[b])

    def idx_wait(j, b):
        pltpu.make_async_copy(src_hbm.at[pl.ds(base(j), CHE)], sv[b], ss[b]).wait()
        pltpu.make_async_copy(dst_hbm.at[pl.ds(base(j), CHE)], dv[b], ds[b]).wait()

    def gather_issue(b):
        pltpu.async_copy(a_hbm.at[sv[b]], av[b], ga[b])
        pltpu.async_copy(b_hbm.at[dv[b]], bv[b], gb[b])

    def store_wait(j, b):
        pltpu.make_async_copy(av[b], he_hbm.at[pl.ds(base(j), CHE)], os[b]).wait()

    idx_issue(0, 0)
    idx_wait(0, 0)
    gather_issue(0)
    idx_issue(1, 1)

    @pl.loop(0, NFE, step=2)
    def _(j0):
        for b in range(2):
            j = j0 + b
            q = 1 - b

            @pl.when(j + 1 < NFE)
            def _():
                idx_wait(j + 1, q)

                @pl.when(j >= 1)
                def _():
                    store_wait(j - 1, q)

                gather_issue(q)

            pltpu.make_async_copy(a_hbm.at[sv[b]], av[b], ga[b]).wait()
            pltpu.make_async_copy(b_hbm.at[dv[b]], bv[b], gb[b]).wait()
            _ew(av[b], av[b], bv[b], CHE, "add")
            pltpu.async_copy(av[b], he_hbm.at[pl.ds(base(j), CHE)], os[b])

            @pl.when(j + 2 < NFE)
            def _():
                idx_issue(j + 2, b)

    store_wait(NFE - 2, 0)
    store_wait(NFE - 1, 1)

    tb = base0 + NFE * CHE
    pltpu.sync_copy(src_hbm.at[pl.ds(tb, TAILE)], sv8)
    pltpu.sync_copy(dst_hbm.at[pl.ds(tb, TAILE)], dv8)
    pltpu.sync_copy(a_hbm.at[sv8], av8)
    pltpu.sync_copy(b_hbm.at[dv8], bv8)
    _ew(av8, av8, bv8, TAILE, "add")
    pltpu.sync_copy(av8, he_hbm.at[pl.ds(tb, TAILE)])


_he_call = functools.partial(
    pl.kernel,
    out_type=jax.ShapeDtypeStruct((E, H), _f32),
    mesh=_mesh,
    scratch_types=[
        [pltpu.VMEM((CHE,), jnp.int32)] * 2,
        [pltpu.VMEM((CHE,), jnp.int32)] * 2,
        [pltpu.VMEM((CHE, H), _f32)] * 2,
        [pltpu.VMEM((CHE, H), _f32)] * 2,
        pltpu.VMEM((TAILE,), jnp.int32), pltpu.VMEM((TAILE,), jnp.int32),
        pltpu.VMEM((TAILE, H), _f32), pltpu.VMEM((TAILE, H), _f32),
        [pltpu.SemaphoreType.DMA] * 2, [pltpu.SemaphoreType.DMA] * 2,
        [pltpu.SemaphoreType.DMA] * 2, [pltpu.SemaphoreType.DMA] * 2,
        [pltpu.SemaphoreType.DMA] * 2,
    ],
)(_he_body)


def _conv_body(h_hbm, src_hbm, dst_hbm, co_hbm, p0_hbm, p1_hbm,
               sv, dv, cv, hv, sv8, dv8, acc,
               ss, ds, cs, gs, ws):
    core = lax.axis_index("c")
    s = lax.axis_index("s")
    w = core * NS + s

    @pl.loop(0, CHC)
    def _(r):
        for g in range(H // 16):
            cv[0].at[pl.ds(r, 1), pl.ds(g * 16, 16)][...] = \
                jnp.zeros((1, 16), _f32)

    for k in range(NPT // CHC):
        pltpu.sync_copy(cv[0], acc.at[pl.ds(s * NPT + k * CHC, CHC)])
    pltpu.sync_copy(cv[0].at[pl.ds(0, NPT % CHC)],
                    acc.at[pl.ds(s * NPT + (NPT // CHC) * CHC, NPT % CHC)])

    @pl.when(s == NS - 1)
    def _():
        pltpu.sync_copy(cv[0].at[pl.ds(0, 16)], acc.at[pl.ds(NS * NPT, 16)])

    plsc.subcore_barrier()

    base0 = w * EPT
    base = lambda j: base0 + j * CHC

    def idx_issue(j, b):
        pltpu.async_copy(src_hbm.at[pl.ds(base(j), CHC)], sv[b], ss[b])
        pltpu.async_copy(dst_hbm.at[pl.ds(base(j), CHC)], dv[b], ds[b])
        pltpu.async_copy(co_hbm.at[pl.ds(base(j), CHC)], cv[b], cs[b])

    def scat_wait(b):
        pltpu.make_async_copy(hv[b], acc.at[dv[b]], ws[b]).wait()

    idx_issue(0, 0)
    pltpu.make_async_copy(src_hbm.at[pl.ds(base(0), CHC)], sv[0], ss[0]).wait()
    pltpu.async_copy(h_hbm.at[sv[0]], hv[0], gs[0])
    idx_issue(1, 1)

    @pl.loop(0, NFC, step=2)
    def _(j0):
        for b in range(2):
            j = j0 + b
            q = 1 - b

            @pl.when(j + 1 < NFC)
            def _():
                pltpu.make_async_copy(
                    src_hbm.at[pl.ds(base(j + 1), CHC)], sv[q], ss[q]).wait()

                @pl.when(j >= 1)
                def _():
                    scat_wait(q)

                pltpu.async_copy(h_hbm.at[sv[q]], hv[q], gs[q])

            pltpu.make_async_copy(h_hbm.at[sv[b]], hv[b], gs[b]).wait()
            pltpu.make_async_copy(
                co_hbm.at[pl.ds(base(j), CHC)], cv[b], cs[b]).wait()
            _ew(hv[b], hv[b], cv[b], CHC, "mul")
            pltpu.make_async_copy(
                dst_hbm.at[pl.ds(base(j), CHC)], dv[b], ds[b]).wait()
            pltpu.async_copy(hv[b], acc.at[dv[b]], ws[b], add=True)

            @pl.when(j + 2 < NFC)
            def _():
                idx_issue(j + 2, b)

    scat_wait(0)
    scat_wait(1)

    tb = base0 + NFC * CHC
    pltpu.sync_copy(src_hbm.at[pl.ds(tb, TAILC)], sv8)
    pltpu.sync_copy(dst_hbm.at[pl.ds(tb, TAILC)], dv8)
    pltpu.sync_copy(co_hbm.at[pl.ds(tb, TAILC)], cv[0].at[pl.ds(0, TAILC)])
    pltpu.sync_copy(h_hbm.at[sv8], hv[0].at[pl.ds(0, TAILC)])
    _ew(hv[0], hv[0], cv[0], TAILC, "mul")
    pltpu.sync_copy(hv[0].at[pl.ds(0, TAILC)], acc.at[dv8], add=True)
    plsc.subcore_barrier()

    def wb(rs, nr):
        @pl.when(core == 0)
        def _():
            pltpu.sync_copy(acc.at[pl.ds(rs, nr)], p0_hbm.at[pl.ds(rs, nr)])

        @pl.when(core == 1)
        def _():
            pltpu.sync_copy(acc.at[pl.ds(rs, nr)], p1_hbm.at[pl.ds(rs, nr)])

    for k in range(NPT // ZR):
        wb(s * NPT + k * ZR, ZR)

    @pl.when(s == NS - 1)
    def _():
        wb(NS * NPT, 16)


_conv_call = functools.partial(
    pl.kernel,
    out_type=[jax.ShapeDtypeStruct((N, H), _f32),
              jax.ShapeDtypeStruct((N, H), _f32)],
    mesh=_mesh,
    scratch_types=[
        [pltpu.VMEM((CHC,), jnp.int32)] * 2,
        [pltpu.VMEM((CHC,), jnp.int32)] * 2,
        [pltpu.VMEM((CHC, H), _f32)] * 2,
        [pltpu.VMEM((CHC, H), _f32)] * 2,
        pltpu.VMEM((TAILC,), jnp.int32), pltpu.VMEM((TAILC,), jnp.int32),
        pltpu.VMEM_SHARED((N, H), _f32),
        [pltpu.SemaphoreType.DMA] * 2, [pltpu.SemaphoreType.DMA] * 2,
        [pltpu.SemaphoreType.DMA] * 2, [pltpu.SemaphoreType.DMA] * 2,
        [pltpu.SemaphoreType.DMA] * 2,
    ],
)(_conv_body)


def _ab_body(z_ref, e1_ref, e2_ref, a_ref, b_ref):
    oh = (z_ref[...] == lax.broadcasted_iota(jnp.int32, (1, NCOLORS), 1)
          ).astype(_f32)
    a_ref[...] = jnp.dot(oh, e1_ref[...], preferred_element_type=_f32,
            precision=lax.Precision.HIGHEST)
    b_ref[...] = jnp.dot(oh, e2_ref[...], preferred_element_type=_f32,
            precision=lax.Precision.HIGHEST)


def _ab(z2, emb1, emb2):
    return pl.pallas_call(
        _ab_body,
        out_shape=[jax.ShapeDtypeStruct((N, H), _f32),
                   jax.ShapeDtypeStruct((N, H), _f32)],
    )(z2, emb1, emb2)


EB = 2000


def _coeff_body(d_ref, cen_ref, ng_ref, he_ref, w1_ref, b1_ref, w2_ref, b2_ref,
                *c_refs):
    dd = d_ref[...]
    diff = dd - cen_ref[...]
    bf = jnp.exp(ng_ref[...] * (diff * diff))
    co = jnp.where(dd < 1.0, 0.5 * (jnp.cos(_PI * dd) + 1.0), 0.0)
    bfc = bf * co
    he = he_ref[...]
    for i in range(L):
        t = _ssp(jnp.dot(bfc, w1_ref[i].astype(jnp.bfloat16),
                         preferred_element_type=_f32) + b1_ref[i])
        filt = jnp.dot(t.astype(jnp.bfloat16), w2_ref[i].astype(jnp.bfloat16),
                       preferred_element_type=_f32) + b2_ref[i]
        c_refs[i][...] = filt * he * co


def _coeff(d2, cen, ng, he, w1, b1, w2, b2):
    blk = lambda i: (i, 0)
    full3 = lambda i: (0, 0, 0)
    full = lambda i: (0, 0)
    return pl.pallas_call(
        _coeff_body,
        grid=(E // EB,),
        in_specs=[
            pl.BlockSpec((EB, 1), blk),
            pl.BlockSpec((1, H), full),
            pl.BlockSpec((1, 1), full),
            pl.BlockSpec((EB, H), blk),
            pl.BlockSpec((L, H, H), full3),
            pl.BlockSpec((L, H), full),
            pl.BlockSpec((L, H, H), full3),
            pl.BlockSpec((L, H), full),
        ],
        out_specs=[pl.BlockSpec((EB, H), blk)] * L,
        out_shape=[jax.ShapeDtypeStruct((E, H), _f32)] * L,
    )(d2, cen, ng, he, w1, b1, w2, b2)


NB = 2000


def _post_body(res, p0_ref, p1_ref, wo_ref, bo_ref, *rest):
    if res:
        h_ref, o_ref = rest
    else:
        (o_ref,) = rest
    agg = p0_ref[...] + p1_ref[...]
    o = _ssp(jnp.dot(agg, wo_ref[...].astype(jnp.bfloat16),
                     preferred_element_type=_f32) + bo_ref[...])
    if res:
        o = o + h_ref[...]
    o_ref[...] = o


def _post(p0, p1, wo, bo, hprev):
    res = hprev is not None
    blk = lambda i: (i, 0)
    full = lambda i: (0, 0)
    in_specs = [
        pl.BlockSpec((NB, H), blk),
        pl.BlockSpec((NB, H), blk),
        pl.BlockSpec((H, H), full),
        pl.BlockSpec((1, H), full),
    ]
    args = [p0, p1, wo, bo]
    if res:
        in_specs.append(pl.BlockSpec((NB, H), blk))
        args.append(hprev)
    return pl.pallas_call(
        functools.partial(_post_body, res),
        grid=(N // NB,),
        in_specs=in_specs,
        out_specs=pl.BlockSpec((NB, H), blk),
        out_shape=jax.ShapeDtypeStruct((N, H), _f32),
    )(*args)


def _readout_body(h_ref, wfc_ref, bfc_ref, wfc2_ref, bfc2_ref, o_ref):
    yn = jnp.dot(h_ref[...].astype(jnp.bfloat16), wfc_ref[...],
                 preferred_element_type=_f32) + bfc_ref[...]
    m = jnp.sum(yn, axis=0, keepdims=True) * np.float32(1.0 / N)
    o_ref[...] = jnp.sum(m * wfc2_ref[...], axis=1, keepdims=True) \
        + bfc2_ref[...]


def _readout(h, wfc, bfc2d, wfc2row, bfc2):
    return pl.pallas_call(
        _readout_body,
        out_shape=jax.ShapeDtypeStruct((1, 1), _f32),
    )(h, wfc, bfc2d, wfc2row, bfc2)


def kernel(x, z, edge_index, d, emb1, emb2, Wf1, bf1, Wf2, bf2, Wo, bo,
           Wfc, bfc, Wfc2, bfc2):
    src = edge_index[0]
    dst = edge_index[1]
    d2 = d.reshape(E, 1)
    z2 = z.astype(jnp.int32).reshape(N, 1)

    centers = jnp.linspace(0.0, 1.0, H)
    gamma = 1.0 / (centers[1] - centers[0]) ** 2
    cen = centers.reshape(1, H)
    ng = (-gamma).reshape(1, 1).astype(_f32)

    a_nodes, b_nodes = _ab(z2, emb1, emb2)
    he = _he_call(src, dst, a_nodes, b_nodes)

    coeffs = _coeff(d2, cen, ng, he, Wf1, bf1, Wf2, bf2)

    h = x
    for i in range(L):
        p0, p1 = _conv_call(h, src, dst, coeffs[i])
        h = _post(p0, p1, Wo[i], bo[i].reshape(1, H), None if i == 0 else h)

    return _readout(h, Wfc, bfc.reshape(1, H), Wfc2.reshape(1, H),
                    bfc2.reshape(1, 1))
